# baseline (speedup 1.0000x reference)
import functools

import numpy as np
import jax
import jax.numpy as jnp
from jax import lax
from jax.experimental import pallas as pl
from jax.experimental.pallas import tpu as pltpu
from jax.experimental.pallas import tpu_sc as plsc

_D = 128
_MAX_LEN = 1000
_B = 1024
_L = 128
_NC = 2
_NS = 16
_NW = _NC * _NS
_N = _B * _L
_PER_W = _N // _NW
_CH = 128
_NCH = _PER_W // _CH


def _make_pe():
    position = np.arange(_MAX_LEN, dtype=np.float32)[:, None]
    div_term = np.exp(
        np.arange(0, _D, 2, dtype=np.float32) * (-np.log(10000.0) / _D))
    pe = np.zeros((_MAX_LEN, _D), dtype=np.float32)
    pe[:, 0::2] = np.sin(position * div_term)
    pe[:, 1::2] = np.cos(position * div_term)
    return jnp.asarray(pe[:_L])


_mesh = plsc.VectorSubcoreMesh(
    core_axis_name="c", subcore_axis_name="s",
    num_cores=_NC, num_subcores=_NS)


@functools.partial(
    pl.kernel,
    out_type=jax.ShapeDtypeStruct((_N, _D), jnp.float32),
    mesh=_mesh,
    scratch_types=[
        pltpu.VMEM((_NCH, _CH), jnp.int32),
        pltpu.VMEM((_CH, _D), jnp.float32),
    ],
)
def _emb_kernel(w_hbm, xr_hbm, pe_hbm, out_hbm, idx_v, rows_v):
    wid = lax.axis_index("s") * _NC + lax.axis_index("c")
    base = wid * _PER_W
    pltpu.sync_copy(xr_hbm.at[wid], idx_v)

    def chunk(i, carry):
        pltpu.sync_copy(pe_hbm, rows_v)
        pltpu.sync_copy(w_hbm.at[idx_v.at[i]], rows_v, add=True)
        pltpu.sync_copy(rows_v, out_hbm.at[pl.ds(base + i * _CH, _CH)])
        return carry

    lax.fori_loop(0, _NCH, chunk, 0)


def kernel(x, W):
    pe = _make_pe()
    xr = x.reshape(_NW, _NCH, _CH)
    out = _emb_kernel(W, xr, pe)
    return out.reshape(_B, _L, _D)


# speedup vs baseline: 1.0355x; 1.0355x over previous
import functools

import numpy as np
import jax
import jax.numpy as jnp
from jax import lax
from jax.experimental import pallas as pl
from jax.experimental.pallas import tpu as pltpu
from jax.experimental.pallas import tpu_sc as plsc

_D = 128
_MAX_LEN = 1000
_B = 1024
_L = 128
_NC = 2
_NS = 16
_NW = _NC * _NS
_N = _B * _L
_PER_W = _N // _NW
_CH = 128
_NCH = _PER_W // _CH
_NB = 4
_NG = _NCH // _NB


def _make_pe():
    position = np.arange(_MAX_LEN, dtype=np.float32)[:, None]
    div_term = np.exp(
        np.arange(0, _D, 2, dtype=np.float32) * (-np.log(10000.0) / _D))
    pe = np.zeros((_MAX_LEN, _D), dtype=np.float32)
    pe[:, 0::2] = np.sin(position * div_term)
    pe[:, 1::2] = np.cos(position * div_term)
    return jnp.asarray(pe[:_L])


_mesh = plsc.VectorSubcoreMesh(
    core_axis_name="c", subcore_axis_name="s",
    num_cores=_NC, num_subcores=_NS)


@functools.partial(
    pl.kernel,
    out_type=jax.ShapeDtypeStruct((_N, _D), jnp.float32),
    mesh=_mesh,
    scratch_types=(
        [pltpu.VMEM((_NCH, _CH), jnp.int32)]
        + [pltpu.VMEM((_CH, _D), jnp.float32) for _ in range(_NB)]
        + [pltpu.SemaphoreType.DMA for _ in range(3 * _NB)]
    ),
)
def _emb_kernel(w_hbm, xr_hbm, pe_hbm, out_hbm, idx_v, *scratch):
    bufs = scratch[:_NB]
    ssems = scratch[_NB:2 * _NB]
    gsems = scratch[2 * _NB:3 * _NB]
    osems = scratch[3 * _NB:]
    wid = lax.axis_index("s") * _NC + lax.axis_index("c")
    base = wid * _PER_W
    pltpu.sync_copy(xr_hbm.at[wid], idx_v)

    for b in range(_NB):
        pltpu.async_copy(pe_hbm, bufs[b], ssems[b])

    def group(g, carry):
        a0 = g * _NB
        for b in range(_NB):
            pltpu.make_async_copy(pe_hbm, bufs[b], ssems[b]).wait()
            pltpu.async_copy(
                w_hbm.at[idx_v.at[a0 + b]], bufs[b], gsems[b], add=True)
        for b in range(_NB):
            pltpu.make_async_copy(
                w_hbm.at[idx_v.at[a0 + b]], bufs[b], gsems[b]).wait()
            pltpu.async_copy(
                bufs[b], out_hbm.at[pl.ds(base + (a0 + b) * _CH, _CH)],
                osems[b])
        for b in range(_NB):
            pltpu.make_async_copy(
                bufs[b], out_hbm.at[pl.ds(base, _CH)], osems[b]).wait()
            pltpu.async_copy(pe_hbm, bufs[b], ssems[b])
        return carry

    lax.fori_loop(0, _NG, group, 0)

    for b in range(_NB):
        pltpu.make_async_copy(pe_hbm, bufs[b], ssems[b]).wait()


def kernel(x, W):
    pe = _make_pe()
    xr = x.reshape(_NW, _NCH, _CH)
    out = _emb_kernel(W, xr, pe)
    return out.reshape(_B, _L, _D)


# speedup vs baseline: 1.8685x; 1.8045x over previous
import functools

import numpy as np
import jax
import jax.numpy as jnp
from jax import lax
from jax.experimental import pallas as pl
from jax.experimental.pallas import tpu as pltpu
from jax.experimental.pallas import tpu_sc as plsc

_D = 128
_MAX_LEN = 1000
_B = 1024
_L = 128
_NC = 2
_NS = 16
_NW = _NC * _NS
_N = _B * _L
_PER_W = _N // _NW
_CH = 128
_NCH = _PER_W // _CH
_NB = 4
_NG = _NCH // _NB


def _make_pe():
    position = np.arange(_MAX_LEN, dtype=np.float32)[:, None]
    div_term = np.exp(
        np.arange(0, _D, 2, dtype=np.float32) * (-np.log(10000.0) / _D))
    pe = np.zeros((_MAX_LEN, _D), dtype=np.float32)
    pe[:, 0::2] = np.sin(position * div_term)
    pe[:, 1::2] = np.cos(position * div_term)
    return jnp.asarray(pe[:_L])


_mesh = plsc.VectorSubcoreMesh(
    core_axis_name="c", subcore_axis_name="s",
    num_cores=_NC, num_subcores=_NS)


@functools.partial(
    pl.kernel,
    out_type=jax.ShapeDtypeStruct((_N, _D), jnp.float32),
    mesh=_mesh,
    scratch_types=(
        [pltpu.VMEM((_NCH, _CH), jnp.int32),
         pltpu.VMEM((_L, _D), jnp.float32)]
        + [pltpu.VMEM((_CH, _D), jnp.float32) for _ in range(_NB)]
        + [pltpu.SemaphoreType.DMA for _ in range(2 * _NB)]
    ),
)
def _emb_kernel(w_hbm, xr_hbm, pe_hbm, out_hbm, idx_v, pe_v, *sc):
    bufs = sc[:_NB]
    gsems = sc[_NB:2 * _NB]
    osems = sc[2 * _NB:]
    wid = lax.axis_index("s") * _NC + lax.axis_index("c")
    base = wid * _PER_W
    pltpu.sync_copy(xr_hbm.at[wid], idx_v)
    pltpu.sync_copy(pe_hbm, pe_v)

    for b in range(_NB):
        pltpu.async_copy(w_hbm.at[idx_v.at[b]], bufs[b], gsems[b])

    def add_pe(buf):
        def row(r, c_):
            for c in range(8):
                s = pl.ds(c * 16, 16)
                buf[r, s] = buf[r, s] + pe_v[r, s]
            return c_
        lax.fori_loop(0, _CH, row, 0)

    def group(g, carry):
        a0 = g * _NB
        for b in range(_NB):
            pltpu.make_async_copy(
                w_hbm.at[idx_v.at[a0 + b]], bufs[b], gsems[b]).wait()
            add_pe(bufs[b])
            pltpu.async_copy(
                bufs[b], out_hbm.at[pl.ds(base + (a0 + b) * _CH, _CH)],
                osems[b])

        @pl.when(g < _NG - 1)
        def _():
            nxt = a0 + _NB
            for b in range(_NB):
                pltpu.make_async_copy(
                    bufs[b], out_hbm.at[pl.ds(base, _CH)], osems[b]).wait()
                pltpu.async_copy(
                    w_hbm.at[idx_v.at[nxt + b]], bufs[b], gsems[b])
        return carry

    lax.fori_loop(0, _NG, group, 0)
    for b in range(_NB):
        pltpu.make_async_copy(
            bufs[b], out_hbm.at[pl.ds(base, _CH)], osems[b]).wait()


def kernel(x, W):
    pe = _make_pe()
    xr = x.reshape(_NW, _NCH, _CH)
    out = _emb_kernel(W, xr, pe)
    return out.reshape(_B, _L, _D)


# speedup vs baseline: 2.3132x; 1.2380x over previous
import functools

import numpy as np
import jax
import jax.numpy as jnp
from jax import lax
from jax.experimental import pallas as pl
from jax.experimental.pallas import tpu as pltpu
from jax.experimental.pallas import tpu_sc as plsc

_D = 128
_MAX_LEN = 1000
_B = 1024
_L = 128
_NC = 2
_NS = 16
_NW = _NC * _NS
_N = _B * _L
_PER_W = _N // _NW
_CH = 128
_NCH = _PER_W // _CH
_NB = 4
_NG = _NCH // _NB


def _make_pe():
    position = np.arange(_MAX_LEN, dtype=np.float32)[:, None]
    div_term = np.exp(
        np.arange(0, _D, 2, dtype=np.float32) * (-np.log(10000.0) / _D))
    pe = np.zeros((_MAX_LEN, _D), dtype=np.float32)
    pe[:, 0::2] = np.sin(position * div_term)
    pe[:, 1::2] = np.cos(position * div_term)
    return jnp.asarray(pe[:_L])


_mesh = plsc.VectorSubcoreMesh(
    core_axis_name="c", subcore_axis_name="s",
    num_cores=_NC, num_subcores=_NS)


@functools.partial(
    pl.kernel,
    out_type=jax.ShapeDtypeStruct((_N, _D), jnp.float32),
    mesh=_mesh,
    scratch_types=(
        [pltpu.VMEM((_NCH, _CH), jnp.int32),
         pltpu.VMEM((_L, _D), jnp.float32)]
        + [pltpu.VMEM((_CH, _D), jnp.float32) for _ in range(_NB)]
        + [pltpu.SemaphoreType.DMA for _ in range(2 * _NB)]
    ),
)
def _emb_kernel(w_hbm, xr_hbm, pe_hbm, out_hbm, idx_v, pe_v, *sc):
    bufs = sc[:_NB]
    gsems = sc[_NB:2 * _NB]
    osems = sc[2 * _NB:]
    wid = lax.axis_index("s") * _NC + lax.axis_index("c")
    base = wid * _PER_W
    pltpu.sync_copy(xr_hbm.at[wid], idx_v)
    pltpu.sync_copy(pe_hbm, pe_v)

    for b in range(_NB):
        pltpu.async_copy(w_hbm.at[idx_v.at[b]], bufs[b], gsems[b])

    def add_pe(buf):
        def rows(r2, c_):
            r = r2 * 2
            for rr in (r, r + 1):
                for c in range(8):
                    s = pl.ds(c * 16, 16)
                    buf[rr, s] = buf[rr, s] + pe_v[rr, s]
            return c_
        lax.fori_loop(0, _CH // 2, rows, 0)

    def group(g, carry):
        a0 = g * _NB
        nxt = a0 + _NB
        more = g < _NG - 1
        for b in range(_NB):
            pltpu.make_async_copy(
                w_hbm.at[idx_v.at[a0 + b]], bufs[b], gsems[b]).wait()
            add_pe(bufs[b])
            pltpu.async_copy(
                bufs[b], out_hbm.at[pl.ds(base + (a0 + b) * _CH, _CH)],
                osems[b])
            if b >= 1:
                @pl.when(more)
                def _(b=b):
                    pltpu.make_async_copy(
                        bufs[b - 1], out_hbm.at[pl.ds(base, _CH)],
                        osems[b - 1]).wait()
                    pltpu.async_copy(
                        w_hbm.at[idx_v.at[nxt + b - 1]], bufs[b - 1],
                        gsems[b - 1])

        @pl.when(more)
        def _():
            pltpu.make_async_copy(
                bufs[_NB - 1], out_hbm.at[pl.ds(base, _CH)],
                osems[_NB - 1]).wait()
            pltpu.async_copy(
                w_hbm.at[idx_v.at[nxt + _NB - 1]], bufs[_NB - 1],
                gsems[_NB - 1])
        return carry

    lax.fori_loop(0, _NG, group, 0)
    for b in range(_NB):
        pltpu.make_async_copy(
            bufs[b], out_hbm.at[pl.ds(base, _CH)], osems[b]).wait()


def kernel(x, W):
    pe = _make_pe()
    xr = x.reshape(_NW, _NCH, _CH)
    out = _emb_kernel(W, xr, pe)
    return out.reshape(_B, _L, _D)
